# Initial kernel scaffold; baseline (speedup 1.0000x reference)
import jax
import jax.numpy as jnp
from jax.experimental import pallas as pl


def kernel(qk, v, random_rotations):
    raise NotImplementedError("write your pallas kernel here")


# speedup vs baseline: 1.6272x; 1.6272x over previous
import functools

import jax
import jax.numpy as jnp
from jax import lax
from jax.experimental import pallas as pl
from jax.experimental.pallas import tpu as pltpu
from jax.experimental.pallas import tpu_sc as plsc

BATCH = 4
SEQ = 4096
DIM = 128
N_HASHES = 8
N_BUCKETS = 64
CHUNK = 128
N_CHUNKS = SEQ * N_HASHES // CHUNK
TOTAL = SEQ * N_HASHES
TB = 512
NEG = -3.4e38


def _hash_body(qk_ref, r_ref, qkn_ref, bkt_ref, rank_ref, offs_ref, carry):
    tb = pl.program_id(1)

    @pl.when(tb == 0)
    def _():
        carry[...] = jnp.zeros_like(carry)

    x = qk_ref[0]
    nrm = jnp.maximum(jnp.sqrt(jnp.sum(x * x, axis=1, keepdims=True)), 1e-12)
    qkn = x / nrm
    qkn_ref[0] = qkn

    rot = jnp.dot(qkn, r_ref[...], preferred_element_type=jnp.float32)
    lane = lax.broadcasted_iota(jnp.int32, (TB, 256), 1)
    grp = lane // 32
    loc = lane % 32

    buckets = []
    for h in range(N_HASHES):
        m = grp == h
        mp = jnp.max(jnp.where(m, rot, NEG), axis=1, keepdims=True)
        ip = jnp.min(jnp.where(m & (rot == mp), loc, 64), axis=1, keepdims=True)
        mn = jnp.max(jnp.where(m, -rot, NEG), axis=1, keepdims=True)
        im = jnp.min(jnp.where(m & (-rot == mn), loc, 64), axis=1, keepdims=True)
        buckets.append(jnp.where(mp >= mn, ip, im + 32))

    lane64 = lax.broadcasted_iota(jnp.int32, (TB, N_BUCKETS), 1)
    ohs = [(b == lane64).astype(jnp.float32) for b in buckets]
    histtok = ohs[0]
    for oh in ohs[1:]:
        histtok = histtok + oh

    ri = lax.broadcasted_iota(jnp.int32, (TB, TB), 0)
    ci = lax.broadcasted_iota(jnp.int32, (TB, TB), 1)
    tri = (ci < ri).astype(jnp.float32)
    cume = jnp.dot(tri, histtok, preferred_element_type=jnp.float32)
    prior = cume + carry[0:1, :]

    ranks = []
    for h in range(N_HASHES):
        csel = jnp.sum(ohs[h] * prior, axis=1, keepdims=True)
        intra = jnp.zeros((TB, 1), jnp.int32)
        for h2 in range(h):
            intra = intra + (buckets[h2] == buckets[h]).astype(jnp.int32)
        ranks.append(csel.astype(jnp.int32) + intra)

    bkt_ref[0] = jnp.concatenate(buckets, axis=1)
    rank_ref[0] = jnp.concatenate(ranks, axis=1)
    new_carry = carry[0:1, :] + jnp.sum(histtok, axis=0, keepdims=True)
    carry[0:1, :] = new_carry
    bi = lax.broadcasted_iota(jnp.int32, (N_BUCKETS, N_BUCKETS), 0)
    bj = lax.broadcasted_iota(jnp.int32, (N_BUCKETS, N_BUCKETS), 1)
    triu = (bi < bj).astype(jnp.float32)
    offs_ref[0] = jnp.dot(new_carry, triu,
                          precision=lax.Precision.HIGHEST,
                          preferred_element_type=jnp.float32).astype(jnp.int32)


def _run_hash(qk, rotations2):
    grid = (BATCH, SEQ // TB)
    return pl.pallas_call(
        _hash_body,
        grid=grid,
        in_specs=[
            pl.BlockSpec((1, TB, DIM), lambda b, t: (b, t, 0)),
            pl.BlockSpec((DIM, 256), lambda b, t: (0, 0)),
        ],
        out_specs=[
            pl.BlockSpec((1, TB, DIM), lambda b, t: (b, t, 0)),
            pl.BlockSpec((1, TB, N_HASHES), lambda b, t: (b, t, 0)),
            pl.BlockSpec((1, TB, N_HASHES), lambda b, t: (b, t, 0)),
            pl.BlockSpec((1, 1, N_BUCKETS), lambda b, t: (b, 0, 0)),
        ],
        out_shape=[
            jax.ShapeDtypeStruct((BATCH, SEQ, DIM), jnp.float32),
            jax.ShapeDtypeStruct((BATCH, SEQ, N_HASHES), jnp.int32),
            jax.ShapeDtypeStruct((BATCH, SEQ, N_HASHES), jnp.int32),
            jax.ShapeDtypeStruct((BATCH, 1, N_BUCKETS), jnp.int32),
        ],
        scratch_shapes=[pltpu.VMEM((8, N_BUCKETS), jnp.float32)],
        compiler_params=pltpu.CompilerParams(
            dimension_semantics=("arbitrary", "arbitrary")),
    )(qk, rotations2)


def _scatter_body(qkn_hbm, v_hbm, bkt_hbm, rank_hbm, offs_hbm,
                  qks_hbm, vs_hbm,
                  bktv, rankv, offs, destv, idxh, qkbuf, vbuf):
    wid = lax.axis_index("s") * 2 + lax.axis_index("c")
    i16 = lax.iota(jnp.int32, 16)
    for b in range(BATCH):
        pltpu.sync_copy(bkt_hbm.at[b, pl.ds(wid * 64, 64)], bktv)
        pltpu.sync_copy(rank_hbm.at[b, pl.ds(wid * 64, 64)], rankv)
        pltpu.sync_copy(offs_hbm.at[b], offs)
        base = jnp.int32(b * TOTAL)
        for g in range(64):
            vb = plsc.load_gather(offs, [bktv[g]])
            destv[pl.ds(g * 16, 16)] = vb + rankv[g] + base
        pltpu.sync_copy(qkn_hbm.at[pl.ds(b * SEQ + wid * 128, 128)], qkbuf)
        pltpu.sync_copy(v_hbm.at[pl.ds(b * SEQ + wid * 128, 128)], vbuf)
        for h in range(N_HASHES):
            for g in range(8):
                jv = (i16 + g * 16) * 8 + h
                idxh[pl.ds(g * 16, 16)] = plsc.load_gather(
                    destv, [jv])
            pltpu.sync_copy(qkbuf, qks_hbm.at[idxh])
            pltpu.sync_copy(vbuf, vs_hbm.at[idxh])


def _run_scatter(qkn_flat, v_flat, bkt_e, rank_e, offs_e):
    mesh = plsc.VectorSubcoreMesh(core_axis_name="c", subcore_axis_name="s")
    fn = functools.partial(
        pl.kernel,
        out_type=[
            jax.ShapeDtypeStruct((BATCH * TOTAL, DIM), jnp.float32),
            jax.ShapeDtypeStruct((BATCH * TOTAL, DIM), jnp.float32),
        ],
        mesh=mesh,
        compiler_params=pltpu.CompilerParams(needs_layout_passes=False),
        scratch_types=[
            pltpu.VMEM((64, 16), jnp.int32),
            pltpu.VMEM((64, 16), jnp.int32),
            pltpu.VMEM((N_BUCKETS,), jnp.int32),
            pltpu.VMEM((1024,), jnp.int32),
            pltpu.VMEM((128,), jnp.int32),
            pltpu.VMEM((128, DIM), jnp.float32),
            pltpu.VMEM((128, DIM), jnp.float32),
        ],
    )(_scatter_body)
    return fn(qkn_flat, v_flat, bkt_e, rank_e, offs_e)


def _attn_body(qc_ref, qp_ref, vc_ref, vp_ref, o_ref):
    scale = DIM ** -0.5
    q = qc_ref[0, 0] * scale
    kc = qc_ref[0, 0]
    kp = qp_ref[0, 0]
    dn = (((1,), (1,)), ((), ()))
    dp = lax.dot_general(q, kp, dn, preferred_element_type=jnp.float32)
    dc = lax.dot_general(q, kc, dn, preferred_element_type=jnp.float32)
    m = jnp.maximum(jnp.max(dp, axis=1, keepdims=True),
                    jnp.max(dc, axis=1, keepdims=True))
    ep = jnp.exp(dp - m)
    ec = jnp.exp(dc - m)
    s = jnp.sum(ep, axis=1, keepdims=True) + jnp.sum(ec, axis=1, keepdims=True)
    o = (jnp.dot(ep, vp_ref[0, 0], preferred_element_type=jnp.float32)
         + jnp.dot(ec, vc_ref[0, 0], preferred_element_type=jnp.float32))
    o_ref[0, 0] = o * (1.0 / (8.0 * s))


def _run_attn(qs, vs):
    grid = (BATCH, N_CHUNKS)
    cur = pl.BlockSpec((1, 1, CHUNK, DIM), lambda b, c: (b, c, 0, 0))
    prev = pl.BlockSpec((1, 1, CHUNK, DIM),
                        lambda b, c: (b, (c + N_CHUNKS - 1) % N_CHUNKS, 0, 0))
    return pl.pallas_call(
        _attn_body,
        grid=grid,
        in_specs=[cur, prev, cur, prev],
        out_specs=pl.BlockSpec((1, 1, CHUNK, DIM), lambda b, c: (b, c, 0, 0)),
        out_shape=jax.ShapeDtypeStruct((BATCH, N_CHUNKS, CHUNK, DIM),
                                       jnp.float32),
        compiler_params=pltpu.CompilerParams(
            dimension_semantics=("parallel", "arbitrary")),
    )(qs, qs, vs, vs)


def _gather_body(outs_hbm, bkt_hbm, rank_hbm, offs_hbm, out_hbm,
                 bktv, rankv, offs, destq, tokq, gbuf, zbuf, spacc):
    wid = lax.axis_index("s") * 2 + lax.axis_index("c")
    i16 = lax.iota(jnp.int32, 16)
    z16 = jnp.zeros((16,), jnp.float32)
    for i in range(32):
        for j in range(DIM // 16):
            zbuf[i, pl.ds(j * 16, 16)] = z16
    t0 = wid * 128
    for b in range(BATCH):
        pltpu.sync_copy(bkt_hbm.at[b, pl.ds(wid * 64, 64)], bktv)
        pltpu.sync_copy(rank_hbm.at[b, pl.ds(wid * 64, 64)], rankv)
        pltpu.sync_copy(offs_hbm.at[b], offs)
        base = jnp.int32(b * TOTAL)
        for r in range(4):
            pltpu.sync_copy(zbuf, spacc.at[pl.ds(t0 + r * 32, 32)])
        for r in range(4):
            for g in range(16):
                row = r * 16 + g
                vb = plsc.load_gather(offs, [bktv[row]])
                destq[pl.ds(g * 16, 16)] = vb + rankv[row] + base
                tokq[pl.ds(g * 16, 16)] = (
                    t0 + r * 32 + 2 * g + (i16 // 8))
            pltpu.sync_copy(outs_hbm.at[destq], gbuf)
            pltpu.sync_copy(gbuf, spacc.at[tokq], add=True)
        pltpu.sync_copy(spacc.at[pl.ds(t0, 128)],
                        out_hbm.at[pl.ds(b * SEQ + t0, 128)])


def _run_gather(outs_flat, bkt_e, rank_e, offs_e):
    mesh = plsc.VectorSubcoreMesh(core_axis_name="c", subcore_axis_name="s")
    fn = functools.partial(
        pl.kernel,
        out_type=jax.ShapeDtypeStruct((BATCH * SEQ, DIM), jnp.float32),
        mesh=mesh,
        compiler_params=pltpu.CompilerParams(needs_layout_passes=False),
        scratch_types=[
            pltpu.VMEM((64, 16), jnp.int32),
            pltpu.VMEM((64, 16), jnp.int32),
            pltpu.VMEM((N_BUCKETS,), jnp.int32),
            pltpu.VMEM((256,), jnp.int32),
            pltpu.VMEM((256,), jnp.int32),
            pltpu.VMEM((256, DIM), jnp.float32),
            pltpu.VMEM((32, DIM), jnp.float32),
            pltpu.VMEM_SHARED((SEQ, DIM), jnp.float32),
        ],
    )(_gather_body)
    return fn(outs_flat, bkt_e, rank_e, offs_e)


def kernel(qk, v, random_rotations):
    rot2 = random_rotations.reshape(DIM, N_HASHES * 32)
    qkn, bkt, rank, offs = _run_hash(qk, rot2)
    bkt_e = bkt.reshape(BATCH, SEQ * N_HASHES // 16, 16)
    rank_e = rank.reshape(BATCH, SEQ * N_HASHES // 16, 16)
    offs_e = offs.reshape(BATCH, N_BUCKETS)
    qks, vs = _run_scatter(qkn.reshape(BATCH * SEQ, DIM),
                           v.reshape(BATCH * SEQ, DIM),
                           bkt_e, rank_e, offs_e)
    outs = _run_attn(qks.reshape(BATCH, N_CHUNKS, CHUNK, DIM),
                     vs.reshape(BATCH, N_CHUNKS, CHUNK, DIM))
    out = _run_gather(outs.reshape(BATCH * TOTAL, DIM),
                      bkt_e, rank_e, offs_e)
    return out.reshape(BATCH, SEQ, DIM)


# speedup vs baseline: 2.3187x; 1.4250x over previous
import functools

import jax
import jax.numpy as jnp
from jax import lax
from jax.experimental import pallas as pl
from jax.experimental.pallas import tpu as pltpu
from jax.experimental.pallas import tpu_sc as plsc

BATCH = 4
SEQ = 4096
DIM = 128
N_HASHES = 8
N_BUCKETS = 64
CHUNK = 128
N_CHUNKS = SEQ * N_HASHES // CHUNK
TOTAL = SEQ * N_HASHES
TB = 512
NEG = -3.4e38


def _hash_body(qk_ref, r_ref, qkn_ref, bkt_ref, rank_ref, offs_ref, carry):
    tb = pl.program_id(1)

    @pl.when(tb == 0)
    def _():
        carry[...] = jnp.zeros_like(carry)

    x = qk_ref[0]
    nrm = jnp.maximum(jnp.sqrt(jnp.sum(x * x, axis=1, keepdims=True)), 1e-12)
    qkn = x / nrm
    qkn_ref[0] = qkn

    loc = lax.broadcasted_iota(jnp.int32, (TB, 32), 1)
    buckets = []
    for h in range(N_HASHES):
        rh = jnp.dot(qkn, r_ref[h], preferred_element_type=jnp.float32)
        mp = jnp.max(rh, axis=1, keepdims=True)
        mnn = jnp.min(rh, axis=1, keepdims=True)
        cond = mp >= -mnn
        target = jnp.where(cond, mp, mnn)
        addv = jnp.where(cond, 0, 32)
        idx = jnp.min(jnp.where(rh == target, loc, 64), axis=1,
                      keepdims=True)
        buckets.append(idx + addv)

    lane64 = lax.broadcasted_iota(jnp.int32, (TB, N_BUCKETS), 1)
    ohs = [(b == lane64).astype(jnp.float32) for b in buckets]
    histtok = ohs[0]
    for oh in ohs[1:]:
        histtok = histtok + oh

    ri = lax.broadcasted_iota(jnp.int32, (TB, TB), 0)
    ci = lax.broadcasted_iota(jnp.int32, (TB, TB), 1)
    tri = (ci < ri).astype(jnp.float32)
    cume = jnp.dot(tri, histtok, preferred_element_type=jnp.float32)
    prior = cume + carry[0:1, :]

    ranks = []
    for h in range(N_HASHES):
        csel = jnp.sum(ohs[h] * prior, axis=1, keepdims=True)
        intra = jnp.zeros((TB, 1), jnp.int32)
        for h2 in range(h):
            intra = intra + (buckets[h2] == buckets[h]).astype(jnp.int32)
        ranks.append(csel.astype(jnp.int32) + intra)

    bkt_ref[0] = jnp.concatenate(buckets, axis=1)
    rank_ref[0] = jnp.concatenate(ranks, axis=1)
    new_carry = carry[0:1, :] + jnp.sum(histtok, axis=0, keepdims=True)
    carry[0:1, :] = new_carry
    bi = lax.broadcasted_iota(jnp.int32, (N_BUCKETS, N_BUCKETS), 0)
    bj = lax.broadcasted_iota(jnp.int32, (N_BUCKETS, N_BUCKETS), 1)
    triu = (bi < bj).astype(jnp.float32)
    offs_ref[0] = jnp.dot(new_carry, triu,
                          precision=lax.Precision.HIGHEST,
                          preferred_element_type=jnp.float32).astype(jnp.int32)


def _run_hash(qk, rotations2):
    grid = (BATCH, SEQ // TB)
    return pl.pallas_call(
        _hash_body,
        grid=grid,
        in_specs=[
            pl.BlockSpec((1, TB, DIM), lambda b, t: (b, t, 0)),
            pl.BlockSpec((N_HASHES, DIM, 32), lambda b, t: (0, 0, 0)),
        ],
        out_specs=[
            pl.BlockSpec((1, TB, DIM), lambda b, t: (b, t, 0)),
            pl.BlockSpec((1, TB, N_HASHES), lambda b, t: (b, t, 0)),
            pl.BlockSpec((1, TB, N_HASHES), lambda b, t: (b, t, 0)),
            pl.BlockSpec((1, 1, N_BUCKETS), lambda b, t: (b, 0, 0)),
        ],
        out_shape=[
            jax.ShapeDtypeStruct((BATCH, SEQ, DIM), jnp.float32),
            jax.ShapeDtypeStruct((BATCH, SEQ, N_HASHES), jnp.int32),
            jax.ShapeDtypeStruct((BATCH, SEQ, N_HASHES), jnp.int32),
            jax.ShapeDtypeStruct((BATCH, 1, N_BUCKETS), jnp.int32),
        ],
        scratch_shapes=[pltpu.VMEM((8, N_BUCKETS), jnp.float32)],
        compiler_params=pltpu.CompilerParams(
            dimension_semantics=("arbitrary", "arbitrary")),
    )(qk, rotations2)


def _scatter_body(qkn_hbm, v_hbm, bkt_hbm, rank_hbm, offs_hbm,
                  qks_hbm, vs_hbm,
                  bktv, rankv, offs, destv, idxh, qkbuf, vbuf):
    wid = lax.axis_index("s") * 2 + lax.axis_index("c")
    i16 = lax.iota(jnp.int32, 16)
    for b in range(BATCH):
        pltpu.sync_copy(bkt_hbm.at[b, pl.ds(wid * 64, 64)], bktv)
        pltpu.sync_copy(rank_hbm.at[b, pl.ds(wid * 64, 64)], rankv)
        pltpu.sync_copy(offs_hbm.at[b], offs)
        base = jnp.int32(b * TOTAL)
        for g in range(64):
            vb = plsc.load_gather(offs, [bktv[g]])
            destv[pl.ds(g * 16, 16)] = vb + rankv[g] + base
        pltpu.sync_copy(qkn_hbm.at[pl.ds(b * SEQ + wid * 128, 128)], qkbuf)
        pltpu.sync_copy(v_hbm.at[pl.ds(b * SEQ + wid * 128, 128)], vbuf)
        for h in range(N_HASHES):
            for g in range(8):
                jv = (i16 + g * 16) * 8 + h
                idxh[pl.ds(g * 16, 16)] = plsc.load_gather(
                    destv, [jv])
            pltpu.sync_copy(qkbuf, qks_hbm.at[idxh])
            pltpu.sync_copy(vbuf, vs_hbm.at[idxh])


def _run_scatter(qkn_flat, v_flat, bkt_e, rank_e, offs_e):
    mesh = plsc.VectorSubcoreMesh(core_axis_name="c", subcore_axis_name="s")
    fn = functools.partial(
        pl.kernel,
        out_type=[
            jax.ShapeDtypeStruct((BATCH * TOTAL, DIM), jnp.float32),
            jax.ShapeDtypeStruct((BATCH * TOTAL, DIM), jnp.float32),
        ],
        mesh=mesh,
        compiler_params=pltpu.CompilerParams(needs_layout_passes=False),
        scratch_types=[
            pltpu.VMEM((64, 16), jnp.int32),
            pltpu.VMEM((64, 16), jnp.int32),
            pltpu.VMEM((N_BUCKETS,), jnp.int32),
            pltpu.VMEM((1024,), jnp.int32),
            pltpu.VMEM((128,), jnp.int32),
            pltpu.VMEM((128, DIM), jnp.float32),
            pltpu.VMEM((128, DIM), jnp.float32),
        ],
    )(_scatter_body)
    return fn(qkn_flat, v_flat, bkt_e, rank_e, offs_e)


def _attn_pair(q, k0, k1, v0, v1):
    dn = (((1,), (1,)), ((), ()))
    qs = q * (DIM ** -0.5)
    d0 = lax.dot_general(qs, k0, dn, preferred_element_type=jnp.float32)
    d1 = lax.dot_general(qs, k1, dn, preferred_element_type=jnp.float32)
    m = jnp.maximum(jnp.max(d0, axis=1, keepdims=True),
                    jnp.max(d1, axis=1, keepdims=True))
    e0 = jnp.exp(d0 - m)
    e1 = jnp.exp(d1 - m)
    s = jnp.sum(e0, axis=1, keepdims=True) + jnp.sum(e1, axis=1, keepdims=True)
    o = (jnp.dot(e0, v0, preferred_element_type=jnp.float32)
         + jnp.dot(e1, v1, preferred_element_type=jnp.float32))
    return o * (1.0 / (8.0 * s))


def _attn_body(qc_ref, qp_ref, vc_ref, vp_ref, o_ref):
    q0 = qc_ref[0, 0]
    q1 = qc_ref[0, 1]
    kp = qp_ref[0, 0]
    v0 = vc_ref[0, 0]
    v1 = vc_ref[0, 1]
    vp = vp_ref[0, 0]
    o_ref[0, 0] = _attn_pair(q0, kp, q0, vp, v0)
    o_ref[0, 1] = _attn_pair(q1, q0, q1, v0, v1)


def _run_attn(qs, vs):
    grid = (BATCH, N_CHUNKS // 2)
    cur = pl.BlockSpec((1, 2, CHUNK, DIM), lambda b, g: (b, g, 0, 0))
    prev = pl.BlockSpec((1, 1, CHUNK, DIM),
                        lambda b, g: (b, (2 * g + N_CHUNKS - 1) % N_CHUNKS,
                                      0, 0))
    return pl.pallas_call(
        _attn_body,
        grid=grid,
        in_specs=[cur, prev, cur, prev],
        out_specs=pl.BlockSpec((1, 2, CHUNK, DIM), lambda b, g: (b, g, 0, 0)),
        out_shape=jax.ShapeDtypeStruct((BATCH, N_CHUNKS, CHUNK, DIM),
                                       jnp.float32),
        compiler_params=pltpu.CompilerParams(
            dimension_semantics=("parallel", "arbitrary")),
    )(qs, qs, vs, vs)


def _gather_body(outs_hbm, bkt_hbm, rank_hbm, offs_hbm, out_hbm,
                 bktv, rankv, offs, destq, tokq, gbuf, zbuf, spacc):
    wid = lax.axis_index("s") * 2 + lax.axis_index("c")
    i16 = lax.iota(jnp.int32, 16)
    z16 = jnp.zeros((16,), jnp.float32)
    for i in range(32):
        for j in range(DIM // 16):
            zbuf[i, pl.ds(j * 16, 16)] = z16
    t0 = wid * 128
    for b in range(BATCH):
        pltpu.sync_copy(bkt_hbm.at[b, pl.ds(wid * 64, 64)], bktv)
        pltpu.sync_copy(rank_hbm.at[b, pl.ds(wid * 64, 64)], rankv)
        pltpu.sync_copy(offs_hbm.at[b], offs)
        base = jnp.int32(b * TOTAL)
        for r in range(4):
            pltpu.sync_copy(zbuf, spacc.at[pl.ds(t0 + r * 32, 32)])
        for r in range(4):
            for g in range(16):
                row = r * 16 + g
                vb = plsc.load_gather(offs, [bktv[row]])
                destq[pl.ds(g * 16, 16)] = vb + rankv[row] + base
                tokq[pl.ds(g * 16, 16)] = (
                    t0 + r * 32 + 2 * g + (i16 // 8))
            pltpu.sync_copy(outs_hbm.at[destq], gbuf)
            pltpu.sync_copy(gbuf, spacc.at[tokq], add=True)
        pltpu.sync_copy(spacc.at[pl.ds(t0, 128)],
                        out_hbm.at[pl.ds(b * SEQ + t0, 128)])


def _run_gather(outs_flat, bkt_e, rank_e, offs_e):
    mesh = plsc.VectorSubcoreMesh(core_axis_name="c", subcore_axis_name="s")
    fn = functools.partial(
        pl.kernel,
        out_type=jax.ShapeDtypeStruct((BATCH * SEQ, DIM), jnp.float32),
        mesh=mesh,
        compiler_params=pltpu.CompilerParams(needs_layout_passes=False),
        scratch_types=[
            pltpu.VMEM((64, 16), jnp.int32),
            pltpu.VMEM((64, 16), jnp.int32),
            pltpu.VMEM((N_BUCKETS,), jnp.int32),
            pltpu.VMEM((256,), jnp.int32),
            pltpu.VMEM((256,), jnp.int32),
            pltpu.VMEM((256, DIM), jnp.float32),
            pltpu.VMEM((32, DIM), jnp.float32),
            pltpu.VMEM_SHARED((SEQ, DIM), jnp.float32),
        ],
    )(_gather_body)
    return fn(outs_flat, bkt_e, rank_e, offs_e)


def kernel(qk, v, random_rotations):
    rot2 = jnp.transpose(random_rotations, (1, 0, 2))
    qkn, bkt, rank, offs = _run_hash(qk, rot2)
    bkt_e = bkt.reshape(BATCH, SEQ * N_HASHES // 16, 16)
    rank_e = rank.reshape(BATCH, SEQ * N_HASHES // 16, 16)
    offs_e = offs.reshape(BATCH, N_BUCKETS)
    qks, vs = _run_scatter(qkn.reshape(BATCH * SEQ, DIM),
                           v.reshape(BATCH * SEQ, DIM),
                           bkt_e, rank_e, offs_e)
    outs = _run_attn(qks.reshape(BATCH, N_CHUNKS, CHUNK, DIM),
                     vs.reshape(BATCH, N_CHUNKS, CHUNK, DIM))
    out = _run_gather(outs.reshape(BATCH * TOTAL, DIM),
                      bkt_e, rank_e, offs_e)
    return out.reshape(BATCH, SEQ, DIM)


# speedup vs baseline: 2.9135x; 1.2565x over previous
import functools

import jax
import jax.numpy as jnp
from jax import lax
from jax.experimental import pallas as pl
from jax.experimental.pallas import tpu as pltpu
from jax.experimental.pallas import tpu_sc as plsc

BATCH = 4
SEQ = 4096
DIM = 128
N_HASHES = 8
N_BUCKETS = 64
CHUNK = 128
N_CHUNKS = SEQ * N_HASHES // CHUNK
TOTAL = SEQ * N_HASHES
TB = 512
NEG = -3.4e38


def _hash_body(qk_ref, r_ref, qkn_ref, bkt_ref, rank_ref, offs_ref, carry):
    tb = pl.program_id(1)

    @pl.when(tb == 0)
    def _():
        carry[...] = jnp.zeros_like(carry)

    x = qk_ref[0]
    nrm = jnp.maximum(jnp.sqrt(jnp.sum(x * x, axis=1, keepdims=True)), 1e-12)
    qkn = x / nrm
    qkn_ref[0] = qkn

    loc = lax.broadcasted_iota(jnp.int32, (TB, 32), 1)
    buckets = []
    for h in range(N_HASHES):
        rh = jnp.dot(qkn, r_ref[h], preferred_element_type=jnp.float32)
        mp = jnp.max(rh, axis=1, keepdims=True)
        mnn = jnp.min(rh, axis=1, keepdims=True)
        cond = mp >= -mnn
        target = jnp.where(cond, mp, mnn)
        addv = jnp.where(cond, 0, 32)
        idx = jnp.min(jnp.where(rh == target, loc, 64), axis=1,
                      keepdims=True)
        buckets.append(idx + addv)

    lane64 = lax.broadcasted_iota(jnp.int32, (TB, N_BUCKETS), 1)
    ohs = [(b == lane64).astype(jnp.float32) for b in buckets]
    histtok = ohs[0]
    for oh in ohs[1:]:
        histtok = histtok + oh

    ri = lax.broadcasted_iota(jnp.int32, (TB, TB), 0)
    ci = lax.broadcasted_iota(jnp.int32, (TB, TB), 1)
    tri = (ci < ri).astype(jnp.float32)
    cume = jnp.dot(tri, histtok, preferred_element_type=jnp.float32)
    prior = cume + carry[0:1, :]

    ranks = []
    for h in range(N_HASHES):
        csel = jnp.sum(ohs[h] * prior, axis=1, keepdims=True)
        intra = jnp.zeros((TB, 1), jnp.int32)
        for h2 in range(h):
            intra = intra + (buckets[h2] == buckets[h]).astype(jnp.int32)
        ranks.append(csel.astype(jnp.int32) + intra)

    bkt_ref[0] = jnp.concatenate(buckets, axis=1)
    rank_ref[0] = jnp.concatenate(ranks, axis=1)
    new_carry = carry[0:1, :] + jnp.sum(histtok, axis=0, keepdims=True)
    carry[0:1, :] = new_carry
    bi = lax.broadcasted_iota(jnp.int32, (N_BUCKETS, N_BUCKETS), 0)
    bj = lax.broadcasted_iota(jnp.int32, (N_BUCKETS, N_BUCKETS), 1)
    triu = (bi < bj).astype(jnp.float32)
    offs_ref[0] = jnp.dot(new_carry, triu,
                          precision=lax.Precision.HIGHEST,
                          preferred_element_type=jnp.float32).astype(jnp.int32)


def _run_hash(qk, rotations2):
    grid = (BATCH, SEQ // TB)
    return pl.pallas_call(
        _hash_body,
        grid=grid,
        in_specs=[
            pl.BlockSpec((1, TB, DIM), lambda b, t: (b, t, 0)),
            pl.BlockSpec((N_HASHES, DIM, 32), lambda b, t: (0, 0, 0)),
        ],
        out_specs=[
            pl.BlockSpec((1, TB, DIM), lambda b, t: (b, t, 0)),
            pl.BlockSpec((1, TB, N_HASHES), lambda b, t: (b, t, 0)),
            pl.BlockSpec((1, TB, N_HASHES), lambda b, t: (b, t, 0)),
            pl.BlockSpec((1, 1, N_BUCKETS), lambda b, t: (b, 0, 0)),
        ],
        out_shape=[
            jax.ShapeDtypeStruct((BATCH, SEQ, DIM), jnp.float32),
            jax.ShapeDtypeStruct((BATCH, SEQ, N_HASHES), jnp.int32),
            jax.ShapeDtypeStruct((BATCH, SEQ, N_HASHES), jnp.int32),
            jax.ShapeDtypeStruct((BATCH, 1, N_BUCKETS), jnp.int32),
        ],
        scratch_shapes=[pltpu.VMEM((8, N_BUCKETS), jnp.float32)],
        compiler_params=pltpu.CompilerParams(
            dimension_semantics=("arbitrary", "arbitrary")),
    )(qk, rotations2)


def _scatter_body(qkn_hbm, v_hbm, bkt_hbm, rank_hbm, offs_hbm,
                  qks_hbm, vs_hbm,
                  bktv, rankv, offs, destv, idxh, qkbuf, vbuf):
    wid = lax.axis_index("s") * 2 + lax.axis_index("c")
    i16 = lax.iota(jnp.int32, 16)
    for b in range(BATCH):
        pltpu.sync_copy(bkt_hbm.at[b, pl.ds(wid * 64, 64)], bktv)
        pltpu.sync_copy(rank_hbm.at[b, pl.ds(wid * 64, 64)], rankv)
        pltpu.sync_copy(offs_hbm.at[b], offs)
        base = jnp.int32(b * TOTAL)
        for g in range(64):
            vb = plsc.load_gather(offs, [bktv[g]])
            destv[pl.ds(g * 16, 16)] = vb + rankv[g] + base
        pltpu.sync_copy(qkn_hbm.at[pl.ds(b * SEQ + wid * 128, 128)], qkbuf)
        pltpu.sync_copy(v_hbm.at[pl.ds(b * SEQ + wid * 128, 128)], vbuf)
        for h in range(N_HASHES):
            for g in range(8):
                jv = (i16 + g * 16) * 8 + h
                idxh[pl.ds(g * 16, 16)] = plsc.load_gather(
                    destv, [jv])
            pltpu.sync_copy(qkbuf, qks_hbm.at[idxh])
            pltpu.sync_copy(vbuf, vs_hbm.at[idxh])


def _run_scatter(qkn_flat, v_flat, bkt_e, rank_e, offs_e):
    mesh = plsc.VectorSubcoreMesh(core_axis_name="c", subcore_axis_name="s")
    fn = functools.partial(
        pl.kernel,
        out_type=[
            jax.ShapeDtypeStruct((BATCH * TOTAL, DIM), jnp.float32),
            jax.ShapeDtypeStruct((BATCH * TOTAL, DIM), jnp.float32),
        ],
        mesh=mesh,
        compiler_params=pltpu.CompilerParams(needs_layout_passes=False),
        scratch_types=[
            pltpu.VMEM((64, 16), jnp.int32),
            pltpu.VMEM((64, 16), jnp.int32),
            pltpu.VMEM((N_BUCKETS,), jnp.int32),
            pltpu.VMEM((1024,), jnp.int32),
            pltpu.VMEM((128,), jnp.int32),
            pltpu.VMEM((128, DIM), jnp.float32),
            pltpu.VMEM((128, DIM), jnp.float32),
        ],
    )(_scatter_body)
    return fn(qkn_flat, v_flat, bkt_e, rank_e, offs_e)


AC = 4


def _attn_body(qc_ref, qp_ref, vc_ref, vp_ref, o_ref):
    dn = (((1,), (1,)), ((), ()))
    scale = DIM ** -0.5
    kcat = jnp.concatenate(
        [qp_ref[0, 0].astype(jnp.bfloat16)]
        + [qc_ref[0, i].astype(jnp.bfloat16) for i in range(AC)], axis=0)
    vcat = jnp.concatenate(
        [vp_ref[0, 0].astype(jnp.bfloat16)]
        + [vc_ref[0, i].astype(jnp.bfloat16) for i in range(AC)], axis=0)
    q = qc_ref[0].reshape(AC * CHUNK, DIM).astype(jnp.bfloat16)
    d = lax.dot_general(q, kcat, dn,
                        preferred_element_type=jnp.float32) * scale
    ri = lax.broadcasted_iota(jnp.int32, (AC * CHUNK, (AC + 1) * CHUNK), 0)
    ci = lax.broadcasted_iota(jnp.int32, (AC * CHUNK, (AC + 1) * CHUNK), 1)
    band = (ci // CHUNK) - (ri // CHUNK)
    dm = jnp.where((band >= 0) & (band <= 1), d, NEG)
    m = jnp.max(dm, axis=1, keepdims=True)
    e = jnp.exp(dm - m)
    s = jnp.sum(e, axis=1, keepdims=True)
    o = jnp.dot(e.astype(jnp.bfloat16), vcat,
                preferred_element_type=jnp.float32)
    o_ref[0] = (o * (1.0 / (8.0 * s))).reshape(AC, CHUNK, DIM)


def _run_attn(qs, vs):
    grid = (BATCH, N_CHUNKS // AC)
    cur = pl.BlockSpec((1, AC, CHUNK, DIM), lambda b, g: (b, g, 0, 0))
    prev = pl.BlockSpec((1, 1, CHUNK, DIM),
                        lambda b, g: (b, (AC * g + N_CHUNKS - 1) % N_CHUNKS,
                                      0, 0))
    return pl.pallas_call(
        _attn_body,
        grid=grid,
        in_specs=[cur, prev, cur, prev],
        out_specs=pl.BlockSpec((1, AC, CHUNK, DIM), lambda b, g: (b, g, 0, 0)),
        out_shape=jax.ShapeDtypeStruct((BATCH, N_CHUNKS, CHUNK, DIM),
                                       jnp.float32),
        compiler_params=pltpu.CompilerParams(
            dimension_semantics=("parallel", "arbitrary")),
    )(qs, qs, vs, vs)


def _gather_body(outs_hbm, bkt_hbm, rank_hbm, offs_hbm, out_hbm,
                 bktv, rankv, offs, destq, tokq, gbuf, zbuf, spacc):
    wid = lax.axis_index("s") * 2 + lax.axis_index("c")
    i16 = lax.iota(jnp.int32, 16)
    z16 = jnp.zeros((16,), jnp.float32)
    for i in range(32):
        for j in range(DIM // 16):
            zbuf[i, pl.ds(j * 16, 16)] = z16
    t0 = wid * 128
    for b in range(BATCH):
        pltpu.sync_copy(bkt_hbm.at[b, pl.ds(wid * 64, 64)], bktv)
        pltpu.sync_copy(rank_hbm.at[b, pl.ds(wid * 64, 64)], rankv)
        pltpu.sync_copy(offs_hbm.at[b], offs)
        base = jnp.int32(b * TOTAL)
        for r in range(4):
            pltpu.sync_copy(zbuf, spacc.at[pl.ds(t0 + r * 32, 32)])
        for r in range(4):
            for g in range(16):
                row = r * 16 + g
                vb = plsc.load_gather(offs, [bktv[row]])
                destq[pl.ds(g * 16, 16)] = vb + rankv[row] + base
                tokq[pl.ds(g * 16, 16)] = (
                    t0 + r * 32 + 2 * g + (i16 // 8))
            pltpu.sync_copy(outs_hbm.at[destq], gbuf)
            pltpu.sync_copy(gbuf, spacc.at[tokq], add=True)
        pltpu.sync_copy(spacc.at[pl.ds(t0, 128)],
                        out_hbm.at[pl.ds(b * SEQ + t0, 128)])


def _run_gather(outs_flat, bkt_e, rank_e, offs_e):
    mesh = plsc.VectorSubcoreMesh(core_axis_name="c", subcore_axis_name="s")
    fn = functools.partial(
        pl.kernel,
        out_type=jax.ShapeDtypeStruct((BATCH * SEQ, DIM), jnp.float32),
        mesh=mesh,
        compiler_params=pltpu.CompilerParams(needs_layout_passes=False),
        scratch_types=[
            pltpu.VMEM((64, 16), jnp.int32),
            pltpu.VMEM((64, 16), jnp.int32),
            pltpu.VMEM((N_BUCKETS,), jnp.int32),
            pltpu.VMEM((256,), jnp.int32),
            pltpu.VMEM((256,), jnp.int32),
            pltpu.VMEM((256, DIM), jnp.float32),
            pltpu.VMEM((32, DIM), jnp.float32),
            pltpu.VMEM_SHARED((SEQ, DIM), jnp.float32),
        ],
    )(_gather_body)
    return fn(outs_flat, bkt_e, rank_e, offs_e)


def kernel(qk, v, random_rotations):
    rot2 = jnp.transpose(random_rotations, (1, 0, 2))
    qkn, bkt, rank, offs = _run_hash(qk, rot2)
    bkt_e = bkt.reshape(BATCH, SEQ * N_HASHES // 16, 16)
    rank_e = rank.reshape(BATCH, SEQ * N_HASHES // 16, 16)
    offs_e = offs.reshape(BATCH, N_BUCKETS)
    qks, vs = _run_scatter(qkn.reshape(BATCH * SEQ, DIM),
                           v.reshape(BATCH * SEQ, DIM),
                           bkt_e, rank_e, offs_e)
    outs = _run_attn(qks.reshape(BATCH, N_CHUNKS, CHUNK, DIM),
                     vs.reshape(BATCH, N_CHUNKS, CHUNK, DIM))
    out = _run_gather(outs.reshape(BATCH * TOTAL, DIM),
                      bkt_e, rank_e, offs_e)
    return out.reshape(BATCH, SEQ, DIM)


# speedup vs baseline: 3.6125x; 1.2399x over previous
import functools

import jax
import jax.numpy as jnp
from jax import lax
from jax.experimental import pallas as pl
from jax.experimental.pallas import tpu as pltpu
from jax.experimental.pallas import tpu_sc as plsc

BATCH = 4
SEQ = 4096
DIM = 128
N_HASHES = 8
N_BUCKETS = 64
CHUNK = 128
N_CHUNKS = SEQ * N_HASHES // CHUNK
TOTAL = SEQ * N_HASHES
TB = 512
NEG = -3.4e38


def _hash_body(qk_ref, r_ref, qkn_ref, bkt_ref, rank_ref, offs_ref, carry):
    tb = pl.program_id(1)

    @pl.when(tb == 0)
    def _():
        carry[...] = jnp.zeros_like(carry)

    x = qk_ref[0]
    nrm = jnp.maximum(jnp.sqrt(jnp.sum(x * x, axis=1, keepdims=True)), 1e-12)
    qkn = x / nrm
    qkn_ref[0] = qkn

    dn = (((1,), (1,)), ((), ()))
    locT = lax.broadcasted_iota(jnp.int32, (32, TB), 0)
    buckets = []
    for h in range(N_HASHES):
        rhT = lax.dot_general(r_ref[h], qkn, dn,
                              preferred_element_type=jnp.float32)
        mp = jnp.max(rhT, axis=0, keepdims=True)
        mnn = jnp.min(rhT, axis=0, keepdims=True)
        cond = mp >= -mnn
        target = jnp.where(cond, mp, mnn)
        addv = jnp.where(cond, 0, 32)
        idx = jnp.min(jnp.where(rhT == target, locT, 64), axis=0,
                      keepdims=True)
        buckets.append(idx + addv)

    sub64 = lax.broadcasted_iota(jnp.int32, (N_BUCKETS, TB), 0)
    ohs = [(b == sub64).astype(jnp.float32) for b in buckets]
    histtok = ohs[0]
    for oh in ohs[1:]:
        histtok = histtok + oh

    ri = lax.broadcasted_iota(jnp.int32, (TB, TB), 0)
    ci = lax.broadcasted_iota(jnp.int32, (TB, TB), 1)
    triu = (ri < ci).astype(jnp.float32)
    cumeT = jnp.dot(histtok, triu, preferred_element_type=jnp.float32)
    priorT = cumeT + carry[:, 0:1]

    ranks = []
    for h in range(N_HASHES):
        csel = jnp.sum(ohs[h] * priorT, axis=0, keepdims=True)
        intra = jnp.zeros((1, TB), jnp.int32)
        for h2 in range(h):
            intra = intra + (buckets[h2] == buckets[h]).astype(jnp.int32)
        ranks.append(csel.astype(jnp.int32) + intra)

    bkt_ref[0] = jnp.concatenate(buckets, axis=0)
    rank_ref[0] = jnp.concatenate(ranks, axis=0)
    new_carry = carry[:, 0:1] + jnp.sum(histtok, axis=1, keepdims=True)
    carry[:, 0:1] = new_carry
    bi = lax.broadcasted_iota(jnp.int32, (N_BUCKETS, N_BUCKETS), 0)
    bj = lax.broadcasted_iota(jnp.int32, (N_BUCKETS, N_BUCKETS), 1)
    triu64 = (bi < bj).astype(jnp.float32)
    offs_ref[0] = lax.dot_general(
        new_carry, triu64, (((0,), (0,)), ((), ())),
        precision=lax.Precision.HIGHEST,
        preferred_element_type=jnp.float32).astype(jnp.int32)


def _run_hash(qk, rotations2):
    grid = (BATCH, SEQ // TB)
    return pl.pallas_call(
        _hash_body,
        grid=grid,
        in_specs=[
            pl.BlockSpec((1, TB, DIM), lambda b, t: (b, t, 0)),
            pl.BlockSpec((N_HASHES, 32, DIM), lambda b, t: (0, 0, 0)),
        ],
        out_specs=[
            pl.BlockSpec((1, TB, DIM), lambda b, t: (b, t, 0)),
            pl.BlockSpec((1, N_HASHES, TB), lambda b, t: (b, 0, t)),
            pl.BlockSpec((1, N_HASHES, TB), lambda b, t: (b, 0, t)),
            pl.BlockSpec((1, 1, N_BUCKETS), lambda b, t: (b, 0, 0)),
        ],
        out_shape=[
            jax.ShapeDtypeStruct((BATCH, SEQ, DIM), jnp.float32),
            jax.ShapeDtypeStruct((BATCH, N_HASHES, SEQ), jnp.int32),
            jax.ShapeDtypeStruct((BATCH, N_HASHES, SEQ), jnp.int32),
            jax.ShapeDtypeStruct((BATCH, 1, N_BUCKETS), jnp.int32),
        ],
        scratch_shapes=[pltpu.VMEM((N_BUCKETS, 128), jnp.float32)],
        compiler_params=pltpu.CompilerParams(
            dimension_semantics=("arbitrary", "arbitrary")),
    )(qk, rotations2)


def _scatter_body(qkn_hbm, v_hbm, bkt_hbm, rank_hbm, offs_hbm,
                  qks_hbm, vs_hbm,
                  bktv, rankv, offs, idxh, qkbuf, vbuf):
    wid = lax.axis_index("s") * 2 + lax.axis_index("c")
    for b in range(BATCH):
        pltpu.sync_copy(offs_hbm.at[b], offs)
        pltpu.sync_copy(qkn_hbm.at[pl.ds(b * SEQ + wid * 128, 128)], qkbuf)
        pltpu.sync_copy(v_hbm.at[pl.ds(b * SEQ + wid * 128, 128)], vbuf)
        base = jnp.int32(b * TOTAL)
        for h in range(N_HASHES):
            pltpu.sync_copy(bkt_hbm.at[b, h, pl.ds(wid * 128, 128)], bktv)
            pltpu.sync_copy(rank_hbm.at[b, h, pl.ds(wid * 128, 128)], rankv)
            for g in range(8):
                vb = bktv[pl.ds(g * 16, 16)]
                off = plsc.load_gather(offs, [vb])
                idxh[pl.ds(g * 16, 16)] = (
                    off + rankv[pl.ds(g * 16, 16)] + base)
            pltpu.sync_copy(qkbuf, qks_hbm.at[idxh])
            pltpu.sync_copy(vbuf, vs_hbm.at[idxh])


def _run_scatter(qkn_flat, v_flat, bkt_ht, rank_ht, offs_e):
    mesh = plsc.VectorSubcoreMesh(core_axis_name="c", subcore_axis_name="s")
    fn = functools.partial(
        pl.kernel,
        out_type=[
            jax.ShapeDtypeStruct((BATCH * TOTAL, DIM), jnp.float32),
            jax.ShapeDtypeStruct((BATCH * TOTAL, DIM), jnp.float32),
        ],
        mesh=mesh,
        compiler_params=pltpu.CompilerParams(needs_layout_passes=False),
        scratch_types=[
            pltpu.VMEM((128,), jnp.int32),
            pltpu.VMEM((128,), jnp.int32),
            pltpu.VMEM((N_BUCKETS,), jnp.int32),
            pltpu.VMEM((128,), jnp.int32),
            pltpu.VMEM((128, DIM), jnp.float32),
            pltpu.VMEM((128, DIM), jnp.float32),
        ],
    )(_scatter_body)
    return fn(qkn_flat, v_flat, bkt_ht, rank_ht, offs_e)


AC = 4


def _attn_body(qc_ref, qp_ref, vc_ref, vp_ref, mask_ref, o_ref):
    dn = (((1,), (1,)), ((), ()))
    scale = DIM ** -0.5
    kcat = jnp.concatenate(
        [qp_ref[0, 0].astype(jnp.bfloat16)]
        + [qc_ref[0, i].astype(jnp.bfloat16) for i in range(AC)], axis=0)
    vcat = jnp.concatenate(
        [vp_ref[0, 0].astype(jnp.bfloat16)]
        + [vc_ref[0, i].astype(jnp.bfloat16) for i in range(AC)], axis=0)
    q = (qc_ref[0].reshape(AC * CHUNK, DIM) * scale).astype(jnp.bfloat16)
    d = lax.dot_general(q, kcat, dn, preferred_element_type=jnp.float32)
    e = jnp.exp(d) * mask_ref[...]
    s = jnp.sum(e, axis=1, keepdims=True)
    o = jnp.dot(e.astype(jnp.bfloat16), vcat,
                preferred_element_type=jnp.float32)
    o_ref[0] = (o * (1.0 / (8.0 * s))).reshape(AC, CHUNK, DIM)


def _band_mask():
    r = jnp.arange(AC * CHUNK)[:, None] // CHUNK
    c = jnp.arange((AC + 1) * CHUNK)[None, :] // CHUNK
    return ((c - r >= 0) & (c - r <= 1)).astype(jnp.float32)


def _run_attn(qs, vs, mask):
    grid = (BATCH, N_CHUNKS // AC)
    cur = pl.BlockSpec((1, AC, CHUNK, DIM), lambda b, g: (b, g, 0, 0))
    prev = pl.BlockSpec((1, 1, CHUNK, DIM),
                        lambda b, g: (b, (AC * g + N_CHUNKS - 1) % N_CHUNKS,
                                      0, 0))
    mspec = pl.BlockSpec((AC * CHUNK, (AC + 1) * CHUNK), lambda b, g: (0, 0))
    return pl.pallas_call(
        _attn_body,
        grid=grid,
        in_specs=[cur, prev, cur, prev, mspec],
        out_specs=pl.BlockSpec((1, AC, CHUNK, DIM), lambda b, g: (b, g, 0, 0)),
        out_shape=jax.ShapeDtypeStruct((BATCH, N_CHUNKS, CHUNK, DIM),
                                       jnp.float32),
        compiler_params=pltpu.CompilerParams(
            dimension_semantics=("parallel", "arbitrary")),
    )(qs, qs, vs, vs, mask)


def _gather_body(outs_hbm, bkt_hbm, rank_hbm, offs_hbm, out_hbm,
                 bktv, rankv, offs, idxh, tokq, gbuf, spacc):
    wid = lax.axis_index("s") * 2 + lax.axis_index("c")
    i16 = lax.iota(jnp.int32, 16)
    t0 = wid * 128
    for g in range(8):
        tokq[pl.ds(g * 16, 16)] = i16 + (t0 + g * 16)
    for b in range(BATCH):
        pltpu.sync_copy(offs_hbm.at[b], offs)
        base = jnp.int32(b * TOTAL)
        for h in range(N_HASHES):
            pltpu.sync_copy(bkt_hbm.at[b, h, pl.ds(wid * 128, 128)], bktv)
            pltpu.sync_copy(rank_hbm.at[b, h, pl.ds(wid * 128, 128)], rankv)
            for g in range(8):
                vb = bktv[pl.ds(g * 16, 16)]
                off = plsc.load_gather(offs, [vb])
                idxh[pl.ds(g * 16, 16)] = (
                    off + rankv[pl.ds(g * 16, 16)] + base)
            pltpu.sync_copy(outs_hbm.at[idxh], gbuf)
            if h == 0:
                pltpu.sync_copy(gbuf, spacc.at[pl.ds(t0, 128)])
            else:
                pltpu.sync_copy(gbuf, spacc.at[tokq], add=True)
        pltpu.sync_copy(spacc.at[pl.ds(t0, 128)],
                        out_hbm.at[pl.ds(b * SEQ + t0, 128)])


def _run_gather(outs_flat, bkt_ht, rank_ht, offs_e):
    mesh = plsc.VectorSubcoreMesh(core_axis_name="c", subcore_axis_name="s")
    fn = functools.partial(
        pl.kernel,
        out_type=jax.ShapeDtypeStruct((BATCH * SEQ, DIM), jnp.float32),
        mesh=mesh,
        compiler_params=pltpu.CompilerParams(needs_layout_passes=False),
        scratch_types=[
            pltpu.VMEM((128,), jnp.int32),
            pltpu.VMEM((128,), jnp.int32),
            pltpu.VMEM((N_BUCKETS,), jnp.int32),
            pltpu.VMEM((128,), jnp.int32),
            pltpu.VMEM((128,), jnp.int32),
            pltpu.VMEM((128, DIM), jnp.float32),
            pltpu.VMEM_SHARED((SEQ, DIM), jnp.float32),
        ],
    )(_gather_body)
    return fn(outs_flat, bkt_ht, rank_ht, offs_e)


def kernel(qk, v, random_rotations):
    rot2 = jnp.transpose(random_rotations, (1, 2, 0))
    qkn, bkt_ht, rank_ht, offs = _run_hash(qk, rot2)
    offs_e = offs.reshape(BATCH, N_BUCKETS)
    qks, vs = _run_scatter(qkn.reshape(BATCH * SEQ, DIM),
                           v.reshape(BATCH * SEQ, DIM),
                           bkt_ht, rank_ht, offs_e)
    outs = _run_attn(qks.reshape(BATCH, N_CHUNKS, CHUNK, DIM),
                     vs.reshape(BATCH, N_CHUNKS, CHUNK, DIM),
                     _band_mask())
    out = _run_gather(outs.reshape(BATCH * TOTAL, DIM),
                      bkt_ht, rank_ht, offs_e)
    return out.reshape(BATCH, SEQ, DIM)


# speedup vs baseline: 3.8650x; 1.0699x over previous
import functools

import jax
import jax.numpy as jnp
from jax import lax
from jax.experimental import pallas as pl
from jax.experimental.pallas import tpu as pltpu
from jax.experimental.pallas import tpu_sc as plsc

BATCH = 4
SEQ = 4096
DIM = 128
N_HASHES = 8
N_BUCKETS = 64
CHUNK = 128
N_CHUNKS = SEQ * N_HASHES // CHUNK
TOTAL = SEQ * N_HASHES
TB = 512
NEG = -3.4e38


def _pack_bf16(x):
    xb = x.astype(jnp.bfloat16)
    lo = lax.bitcast_convert_type(xb[:, :64], jnp.uint16).astype(jnp.uint32)
    hi = lax.bitcast_convert_type(xb[:, 64:], jnp.uint16).astype(jnp.uint32)
    return lax.bitcast_convert_type(lo | (hi << 16), jnp.float32)


def _unpack_bf16(p):
    u = lax.bitcast_convert_type(p, jnp.uint32)
    lo = lax.bitcast_convert_type((u & 0xFFFF).astype(jnp.uint16),
                                  jnp.bfloat16)
    hi = lax.bitcast_convert_type((u >> 16).astype(jnp.uint16), jnp.bfloat16)
    return jnp.concatenate([lo, hi], axis=-1)


def _hash_body(qk_ref, v_ref, r_ref, qvp_ref, bkt_ref, rank_ref,
               offs_ref, carry):
    tb = pl.program_id(1)

    @pl.when(tb == 0)
    def _():
        carry[...] = jnp.zeros_like(carry)

    x = qk_ref[0]
    nrm = jnp.maximum(jnp.sqrt(jnp.sum(x * x, axis=1, keepdims=True)), 1e-12)
    qkn = x / nrm
    qvp_ref[0] = jnp.concatenate(
        [_pack_bf16(qkn), _pack_bf16(v_ref[0])], axis=1)

    dn = (((1,), (1,)), ((), ()))
    locT = lax.broadcasted_iota(jnp.int32, (32, TB), 0)
    buckets = []
    for h in range(N_HASHES):
        rhT = lax.dot_general(r_ref[h], qkn, dn,
                              preferred_element_type=jnp.float32)
        mp = jnp.max(rhT, axis=0, keepdims=True)
        mnn = jnp.min(rhT, axis=0, keepdims=True)
        cond = mp >= -mnn
        target = jnp.where(cond, mp, mnn)
        addv = jnp.where(cond, 0, 32)
        idx = jnp.min(jnp.where(rhT == target, locT, 64), axis=0,
                      keepdims=True)
        buckets.append(idx + addv)

    sub64 = lax.broadcasted_iota(jnp.int32, (N_BUCKETS, TB), 0)
    ohs = [(b == sub64).astype(jnp.float32) for b in buckets]
    histtok = ohs[0]
    for oh in ohs[1:]:
        histtok = histtok + oh

    ri = lax.broadcasted_iota(jnp.int32, (TB, TB), 0)
    ci = lax.broadcasted_iota(jnp.int32, (TB, TB), 1)
    triu = (ri < ci).astype(jnp.float32)
    cumeT = jnp.dot(histtok, triu, preferred_element_type=jnp.float32)
    priorT = cumeT + carry[:, 0:1]

    ranks = []
    for h in range(N_HASHES):
        csel = jnp.sum(ohs[h] * priorT, axis=0, keepdims=True)
        intra = jnp.zeros((1, TB), jnp.int32)
        for h2 in range(h):
            intra = intra + (buckets[h2] == buckets[h]).astype(jnp.int32)
        ranks.append(csel.astype(jnp.int32) + intra)

    bkt_ref[0] = jnp.concatenate(buckets, axis=0)
    rank_ref[0] = jnp.concatenate(ranks, axis=0)
    new_carry = carry[:, 0:1] + jnp.sum(histtok, axis=1, keepdims=True)
    carry[:, 0:1] = new_carry
    bi = lax.broadcasted_iota(jnp.int32, (N_BUCKETS, N_BUCKETS), 0)
    bj = lax.broadcasted_iota(jnp.int32, (N_BUCKETS, N_BUCKETS), 1)
    triu64 = (bi < bj).astype(jnp.float32)
    offs_ref[0] = lax.dot_general(
        new_carry, triu64, (((0,), (0,)), ((), ())),
        precision=lax.Precision.HIGHEST,
        preferred_element_type=jnp.float32).astype(jnp.int32)


def _run_hash(qk, v, rotations2):
    grid = (BATCH, SEQ // TB)
    return pl.pallas_call(
        _hash_body,
        grid=grid,
        in_specs=[
            pl.BlockSpec((1, TB, DIM), lambda b, t: (b, t, 0)),
            pl.BlockSpec((1, TB, DIM), lambda b, t: (b, t, 0)),
            pl.BlockSpec((N_HASHES, 32, DIM), lambda b, t: (0, 0, 0)),
        ],
        out_specs=[
            pl.BlockSpec((1, TB, DIM), lambda b, t: (b, t, 0)),
            pl.BlockSpec((1, N_HASHES, TB), lambda b, t: (b, 0, t)),
            pl.BlockSpec((1, N_HASHES, TB), lambda b, t: (b, 0, t)),
            pl.BlockSpec((1, 1, N_BUCKETS), lambda b, t: (b, 0, 0)),
        ],
        out_shape=[
            jax.ShapeDtypeStruct((BATCH, SEQ, DIM), jnp.float32),
            jax.ShapeDtypeStruct((BATCH, N_HASHES, SEQ), jnp.int32),
            jax.ShapeDtypeStruct((BATCH, N_HASHES, SEQ), jnp.int32),
            jax.ShapeDtypeStruct((BATCH, 1, N_BUCKETS), jnp.int32),
        ],
        scratch_shapes=[pltpu.VMEM((N_BUCKETS, 128), jnp.float32)],
        compiler_params=pltpu.CompilerParams(
            dimension_semantics=("arbitrary", "arbitrary")),
    )(qk, v, rotations2)


def _scatter_body(qv_hbm, bkt_hbm, rank_hbm, offs_hbm, qvs_hbm,
                  bktv, rankv, offs, idxh, qvbuf):
    wid = lax.axis_index("s") * 2 + lax.axis_index("c")
    for b in range(BATCH):
        pltpu.sync_copy(offs_hbm.at[b], offs)
        pltpu.sync_copy(qv_hbm.at[pl.ds(b * SEQ + wid * 128, 128)], qvbuf)
        base = jnp.int32(b * TOTAL)
        for h in range(N_HASHES):
            pltpu.sync_copy(bkt_hbm.at[b, h, pl.ds(wid * 128, 128)], bktv)
            pltpu.sync_copy(rank_hbm.at[b, h, pl.ds(wid * 128, 128)], rankv)
            for g in range(8):
                vb = bktv[pl.ds(g * 16, 16)]
                off = plsc.load_gather(offs, [vb])
                idxh[pl.ds(g * 16, 16)] = (
                    off + rankv[pl.ds(g * 16, 16)] + base)
            pltpu.sync_copy(qvbuf, qvs_hbm.at[idxh])


def _run_scatter(qv_flat, bkt_ht, rank_ht, offs_e):
    mesh = plsc.VectorSubcoreMesh(core_axis_name="c", subcore_axis_name="s")
    fn = functools.partial(
        pl.kernel,
        out_type=jax.ShapeDtypeStruct((BATCH * TOTAL, DIM), jnp.float32),
        mesh=mesh,
        compiler_params=pltpu.CompilerParams(needs_layout_passes=False),
        scratch_types=[
            pltpu.VMEM((128,), jnp.int32),
            pltpu.VMEM((128,), jnp.int32),
            pltpu.VMEM((N_BUCKETS,), jnp.int32),
            pltpu.VMEM((128,), jnp.int32),
            pltpu.VMEM((128, DIM), jnp.float32),
        ],
    )(_scatter_body)
    return fn(qv_flat, bkt_ht, rank_ht, offs_e)


AC = 4


def _attn_body(qvc_ref, qvp_ref, mask_ref, o_ref):
    dn = (((1,), (1,)), ((), ()))
    scale = DIM ** -0.5
    qvcat = jnp.concatenate([qvp_ref[0, 0]]
                            + [qvc_ref[0, i] for i in range(AC)], axis=0)
    kcat = _unpack_bf16(qvcat[:, :DIM // 2])
    vcat = _unpack_bf16(qvcat[:, DIM // 2:])
    q = _unpack_bf16(
        qvc_ref[0, :, :, :DIM // 2].reshape(AC * CHUNK, DIM // 2)
    ) * jnp.bfloat16(scale)
    d = lax.dot_general(q, kcat, dn, preferred_element_type=jnp.float32)
    e = jnp.exp(d) * mask_ref[...]
    s = jnp.sum(e, axis=1, keepdims=True)
    o = jnp.dot(e.astype(jnp.bfloat16), vcat,
                preferred_element_type=jnp.float32)
    o_ref[0] = (o * (1.0 / (8.0 * s))).reshape(AC, CHUNK, DIM)


def _band_mask():
    r = jnp.arange(AC * CHUNK)[:, None] // CHUNK
    c = jnp.arange((AC + 1) * CHUNK)[None, :] // CHUNK
    return ((c - r >= 0) & (c - r <= 1)).astype(jnp.float32)


def _run_attn(qvs, mask):
    grid = (BATCH, N_CHUNKS // AC)
    cur = pl.BlockSpec((1, AC, CHUNK, DIM), lambda b, g: (b, g, 0, 0))
    prev = pl.BlockSpec((1, 1, CHUNK, DIM),
                        lambda b, g: (b, (AC * g + N_CHUNKS - 1) % N_CHUNKS,
                                      0, 0))
    mspec = pl.BlockSpec((AC * CHUNK, (AC + 1) * CHUNK), lambda b, g: (0, 0))
    return pl.pallas_call(
        _attn_body,
        grid=grid,
        in_specs=[cur, prev, mspec],
        out_specs=pl.BlockSpec((1, AC, CHUNK, DIM), lambda b, g: (b, g, 0, 0)),
        out_shape=jax.ShapeDtypeStruct((BATCH, N_CHUNKS, CHUNK, DIM),
                                       jnp.float32),
        compiler_params=pltpu.CompilerParams(
            dimension_semantics=("parallel", "arbitrary")),
    )(qvs, qvs, mask)


def _gather_body(outs_hbm, bkt_hbm, rank_hbm, offs_hbm, out_hbm,
                 bktv, rankv, offs, idxh, tokq, gbuf, spacc):
    wid = lax.axis_index("s") * 2 + lax.axis_index("c")
    i16 = lax.iota(jnp.int32, 16)
    t0 = wid * 128
    for g in range(8):
        tokq[pl.ds(g * 16, 16)] = i16 + (t0 + g * 16)
    for b in range(BATCH):
        pltpu.sync_copy(offs_hbm.at[b], offs)
        base = jnp.int32(b * TOTAL)
        for h in range(N_HASHES):
            pltpu.sync_copy(bkt_hbm.at[b, h, pl.ds(wid * 128, 128)], bktv)
            pltpu.sync_copy(rank_hbm.at[b, h, pl.ds(wid * 128, 128)], rankv)
            for g in range(8):
                vb = bktv[pl.ds(g * 16, 16)]
                off = plsc.load_gather(offs, [vb])
                idxh[pl.ds(g * 16, 16)] = (
                    off + rankv[pl.ds(g * 16, 16)] + base)
            pltpu.sync_copy(outs_hbm.at[idxh], gbuf)
            if h == 0:
                pltpu.sync_copy(gbuf, spacc.at[pl.ds(t0, 128)])
            else:
                pltpu.sync_copy(gbuf, spacc.at[tokq], add=True)
        pltpu.sync_copy(spacc.at[pl.ds(t0, 128)],
                        out_hbm.at[pl.ds(b * SEQ + t0, 128)])


def _run_gather(outs_flat, bkt_ht, rank_ht, offs_e):
    mesh = plsc.VectorSubcoreMesh(core_axis_name="c", subcore_axis_name="s")
    fn = functools.partial(
        pl.kernel,
        out_type=jax.ShapeDtypeStruct((BATCH * SEQ, DIM), jnp.float32),
        mesh=mesh,
        compiler_params=pltpu.CompilerParams(needs_layout_passes=False),
        scratch_types=[
            pltpu.VMEM((128,), jnp.int32),
            pltpu.VMEM((128,), jnp.int32),
            pltpu.VMEM((N_BUCKETS,), jnp.int32),
            pltpu.VMEM((128,), jnp.int32),
            pltpu.VMEM((128,), jnp.int32),
            pltpu.VMEM((128, DIM), jnp.float32),
            pltpu.VMEM_SHARED((SEQ, DIM), jnp.float32),
        ],
    )(_gather_body)
    return fn(outs_flat, bkt_ht, rank_ht, offs_e)


def kernel(qk, v, random_rotations):
    rot2 = jnp.transpose(random_rotations, (1, 2, 0))
    qvp, bkt_ht, rank_ht, offs = _run_hash(qk, v, rot2)
    offs_e = offs.reshape(BATCH, N_BUCKETS)
    qvs = _run_scatter(qvp.reshape(BATCH * SEQ, DIM),
                       bkt_ht, rank_ht, offs_e)
    outs = _run_attn(qvs.reshape(BATCH, N_CHUNKS, CHUNK, DIM), _band_mask())
    out = _run_gather(outs.reshape(BATCH * TOTAL, DIM),
                      bkt_ht, rank_ht, offs_e)
    return out.reshape(BATCH, SEQ, DIM)


# speedup vs baseline: 4.3849x; 1.1345x over previous
import functools

import jax
import jax.numpy as jnp
from jax import lax
from jax.experimental import pallas as pl
from jax.experimental.pallas import tpu as pltpu
from jax.experimental.pallas import tpu_sc as plsc

BATCH = 4
SEQ = 4096
DIM = 128
N_HASHES = 8
N_BUCKETS = 64
CHUNK = 128
N_CHUNKS = SEQ * N_HASHES // CHUNK
TOTAL = SEQ * N_HASHES
TB = 512
NEG = -3.4e38


def _pack_bf16(x):
    xb = x.astype(jnp.bfloat16)
    lo = lax.bitcast_convert_type(xb[:, :64], jnp.uint16).astype(jnp.uint32)
    hi = lax.bitcast_convert_type(xb[:, 64:], jnp.uint16).astype(jnp.uint32)
    return lax.bitcast_convert_type(lo | (hi << 16), jnp.float32)


def _unpack_bf16(p):
    u = lax.bitcast_convert_type(p, jnp.uint32)
    lo = lax.bitcast_convert_type((u & 0xFFFF).astype(jnp.uint16),
                                  jnp.bfloat16)
    hi = lax.bitcast_convert_type((u >> 16).astype(jnp.uint16), jnp.bfloat16)
    return jnp.concatenate([lo, hi], axis=-1)


def _hash_body(qk_ref, v_ref, r_ref, qvp_ref, bkt_ref, rank_ref,
               offs_ref, carry):
    tb = pl.program_id(1)

    @pl.when(tb == 0)
    def _():
        carry[...] = jnp.zeros_like(carry)

    x = qk_ref[0]
    nrm = jnp.maximum(jnp.sqrt(jnp.sum(x * x, axis=1, keepdims=True)), 1e-12)
    qkn = x / nrm
    qvp_ref[0] = jnp.concatenate(
        [_pack_bf16(qkn), _pack_bf16(v_ref[0])], axis=1)

    dn = (((1,), (1,)), ((), ()))
    locT = lax.broadcasted_iota(jnp.int32, (32, TB), 0)
    buckets = []
    for h in range(N_HASHES):
        rhT = lax.dot_general(r_ref[h], qkn, dn,
                              preferred_element_type=jnp.float32)
        mp = jnp.max(rhT, axis=0, keepdims=True)
        mnn = jnp.min(rhT, axis=0, keepdims=True)
        cond = mp >= -mnn
        target = jnp.where(cond, mp, mnn)
        addv = jnp.where(cond, 0, 32)
        idx = jnp.min(jnp.where(rhT == target, locT, 64), axis=0,
                      keepdims=True)
        buckets.append(idx + addv)

    sub64 = lax.broadcasted_iota(jnp.int32, (N_BUCKETS, TB), 0)
    ohs = [(b == sub64).astype(jnp.float32) for b in buckets]
    histtok = ohs[0]
    for oh in ohs[1:]:
        histtok = histtok + oh

    ri = lax.broadcasted_iota(jnp.int32, (TB, TB), 0)
    ci = lax.broadcasted_iota(jnp.int32, (TB, TB), 1)
    triu = (ri < ci).astype(jnp.float32)
    cumeT = jnp.dot(histtok, triu, preferred_element_type=jnp.float32)
    priorT = cumeT + carry[:, 0:1]

    ranks = []
    for h in range(N_HASHES):
        csel = jnp.sum(ohs[h] * priorT, axis=0, keepdims=True)
        intra = jnp.zeros((1, TB), jnp.int32)
        for h2 in range(h):
            intra = intra + (buckets[h2] == buckets[h]).astype(jnp.int32)
        ranks.append(csel.astype(jnp.int32) + intra)

    bkt_ref[0] = jnp.concatenate(buckets, axis=0)
    rank_ref[0] = jnp.concatenate(ranks, axis=0)
    new_carry = carry[:, 0:1] + jnp.sum(histtok, axis=1, keepdims=True)
    carry[:, 0:1] = new_carry
    bi = lax.broadcasted_iota(jnp.int32, (N_BUCKETS, N_BUCKETS), 0)
    bj = lax.broadcasted_iota(jnp.int32, (N_BUCKETS, N_BUCKETS), 1)
    triu64 = (bi < bj).astype(jnp.float32)
    offs_ref[0] = lax.dot_general(
        new_carry, triu64, (((0,), (0,)), ((), ())),
        precision=lax.Precision.HIGHEST,
        preferred_element_type=jnp.float32).astype(jnp.int32)


def _run_hash(qk, v, rotations2):
    grid = (BATCH, SEQ // TB)
    return pl.pallas_call(
        _hash_body,
        grid=grid,
        in_specs=[
            pl.BlockSpec((1, TB, DIM), lambda b, t: (b, t, 0)),
            pl.BlockSpec((1, TB, DIM), lambda b, t: (b, t, 0)),
            pl.BlockSpec((N_HASHES, 32, DIM), lambda b, t: (0, 0, 0)),
        ],
        out_specs=[
            pl.BlockSpec((1, TB, DIM), lambda b, t: (b, t, 0)),
            pl.BlockSpec((1, N_HASHES, TB), lambda b, t: (b, 0, t)),
            pl.BlockSpec((1, N_HASHES, TB), lambda b, t: (b, 0, t)),
            pl.BlockSpec((1, 1, N_BUCKETS), lambda b, t: (b, 0, 0)),
        ],
        out_shape=[
            jax.ShapeDtypeStruct((BATCH, SEQ, DIM), jnp.float32),
            jax.ShapeDtypeStruct((BATCH, N_HASHES, SEQ), jnp.int32),
            jax.ShapeDtypeStruct((BATCH, N_HASHES, SEQ), jnp.int32),
            jax.ShapeDtypeStruct((BATCH, 1, N_BUCKETS), jnp.int32),
        ],
        scratch_shapes=[pltpu.VMEM((N_BUCKETS, 128), jnp.float32)],
        compiler_params=pltpu.CompilerParams(
            dimension_semantics=("arbitrary", "arbitrary")),
    )(qk, v, rotations2)


def _scatter_body(qv_hbm, bkt_hbm, rank_hbm, offs_hbm, qvs_hbm,
                  bktv, rankv, offs, idxh8, qvbuf, ssem):
    wid = lax.axis_index("s") * 2 + lax.axis_index("c")
    pltpu.sync_copy(offs_hbm, offs)
    for b in range(BATCH):
        pltpu.sync_copy(qv_hbm.at[pl.ds(b * SEQ + wid * 128, 128)], qvbuf)
        base = jnp.int32(b * TOTAL)
        boff = jnp.int32(b * N_BUCKETS)
        handles = []
        for h in range(N_HASHES):
            pltpu.sync_copy(bkt_hbm.at[b, h, pl.ds(wid * 128, 128)], bktv)
            pltpu.sync_copy(rank_hbm.at[b, h, pl.ds(wid * 128, 128)], rankv)
            for g in range(8):
                vb = bktv[pl.ds(g * 16, 16)] + boff
                off = plsc.load_gather(offs, [vb])
                idxh8[h, pl.ds(g * 16, 16)] = (
                    off + rankv[pl.ds(g * 16, 16)] + base)
            handles.append(
                pltpu.async_copy(qvbuf, qvs_hbm.at[idxh8.at[h]], ssem))
        for hd in handles:
            hd.wait()


def _run_scatter(qv_flat, bkt_ht, rank_ht, offs_e):
    mesh = plsc.VectorSubcoreMesh(core_axis_name="c", subcore_axis_name="s")
    fn = functools.partial(
        pl.kernel,
        out_type=jax.ShapeDtypeStruct((BATCH * TOTAL, DIM), jnp.float32),
        mesh=mesh,
        compiler_params=pltpu.CompilerParams(needs_layout_passes=False),
        scratch_types=[
            pltpu.VMEM((128,), jnp.int32),
            pltpu.VMEM((128,), jnp.int32),
            pltpu.VMEM((BATCH * N_BUCKETS,), jnp.int32),
            pltpu.VMEM((N_HASHES, 128), jnp.int32),
            pltpu.VMEM((128, DIM), jnp.float32),
            pltpu.SemaphoreType.DMA,
        ],
    )(_scatter_body)
    return fn(qv_flat, bkt_ht, rank_ht, offs_e.reshape(-1))


AC = 4


def _attn_body(qvc_ref, qvp_ref, mask_ref, o_ref):
    dn = (((1,), (1,)), ((), ()))
    scale = DIM ** -0.5
    qvcat = jnp.concatenate([qvp_ref[0, 0]]
                            + [qvc_ref[0, i] for i in range(AC)], axis=0)
    kcat = _unpack_bf16(qvcat[:, :DIM // 2])
    vcat = _unpack_bf16(qvcat[:, DIM // 2:])
    q = _unpack_bf16(
        qvc_ref[0, :, :, :DIM // 2].reshape(AC * CHUNK, DIM // 2)
    ) * jnp.bfloat16(scale)
    d = lax.dot_general(q, kcat, dn, preferred_element_type=jnp.float32)
    e = jnp.exp(d) * mask_ref[...]
    s = jnp.sum(e, axis=1, keepdims=True)
    o = jnp.dot(e.astype(jnp.bfloat16), vcat,
                preferred_element_type=jnp.float32)
    o_ref[0] = (o * (1.0 / (8.0 * s))).reshape(AC, CHUNK, DIM)


def _band_mask():
    r = jnp.arange(AC * CHUNK)[:, None] // CHUNK
    c = jnp.arange((AC + 1) * CHUNK)[None, :] // CHUNK
    return ((c - r >= 0) & (c - r <= 1)).astype(jnp.float32)


def _run_attn(qvs, mask):
    grid = (BATCH, N_CHUNKS // AC)
    cur = pl.BlockSpec((1, AC, CHUNK, DIM), lambda b, g: (b, g, 0, 0))
    prev = pl.BlockSpec((1, 1, CHUNK, DIM),
                        lambda b, g: (b, (AC * g + N_CHUNKS - 1) % N_CHUNKS,
                                      0, 0))
    mspec = pl.BlockSpec((AC * CHUNK, (AC + 1) * CHUNK), lambda b, g: (0, 0))
    return pl.pallas_call(
        _attn_body,
        grid=grid,
        in_specs=[cur, prev, mspec],
        out_specs=pl.BlockSpec((1, AC, CHUNK, DIM), lambda b, g: (b, g, 0, 0)),
        out_shape=jax.ShapeDtypeStruct((BATCH, N_CHUNKS, CHUNK, DIM),
                                       jnp.float32),
        compiler_params=pltpu.CompilerParams(
            dimension_semantics=("parallel", "arbitrary")),
    )(qvs, qvs, mask)


def _gather_body(outs_hbm, bkt_hbm, rank_hbm, offs_hbm, out_hbm,
                 bktv, rankv, offs, idxh8, tokq, gbuf2, spacc, gsem, asem):
    wid = lax.axis_index("s") * 2 + lax.axis_index("c")
    i16 = lax.iota(jnp.int32, 16)
    t0 = wid * 128
    for g in range(8):
        tokq[pl.ds(g * 16, 16)] = i16 + (t0 + g * 16)
    pltpu.sync_copy(offs_hbm, offs)
    for b in range(BATCH):
        base = jnp.int32(b * TOTAL)
        boff = jnp.int32(b * N_BUCKETS)

        def build_idx(h):
            pltpu.sync_copy(bkt_hbm.at[b, h, pl.ds(wid * 128, 128)], bktv)
            pltpu.sync_copy(rank_hbm.at[b, h, pl.ds(wid * 128, 128)], rankv)
            for g in range(8):
                vb = bktv[pl.ds(g * 16, 16)] + boff
                off = plsc.load_gather(offs, [vb])
                idxh8[h, pl.ds(g * 16, 16)] = (
                    off + rankv[pl.ds(g * 16, 16)] + base)

        build_idx(0)
        gh = [None] * N_HASHES
        gh[0] = pltpu.async_copy(outs_hbm.at[idxh8.at[0]], gbuf2.at[0], gsem)
        adds = []
        for h in range(N_HASHES):
            if h + 1 < N_HASHES:
                if h >= 2:
                    adds[h - 2].wait()
                build_idx(h + 1)
                gh[h + 1] = pltpu.async_copy(
                    outs_hbm.at[idxh8.at[h + 1]], gbuf2.at[(h + 1) % 2], gsem)
            gh[h].wait()
            if h == 0:
                pltpu.async_copy(gbuf2.at[0], spacc.at[pl.ds(t0, 128)],
                                 asem).wait()
            else:
                adds.append(pltpu.async_copy(gbuf2.at[h % 2],
                                             spacc.at[tokq], asem, add=True))
        for hd in adds[max(0, N_HASHES - 3):]:
            hd.wait()
        pltpu.sync_copy(spacc.at[pl.ds(t0, 128)],
                        out_hbm.at[pl.ds(b * SEQ + t0, 128)])


def _run_gather(outs_flat, bkt_ht, rank_ht, offs_e):
    mesh = plsc.VectorSubcoreMesh(core_axis_name="c", subcore_axis_name="s")
    fn = functools.partial(
        pl.kernel,
        out_type=jax.ShapeDtypeStruct((BATCH * SEQ, DIM), jnp.float32),
        mesh=mesh,
        compiler_params=pltpu.CompilerParams(needs_layout_passes=False),
        scratch_types=[
            pltpu.VMEM((128,), jnp.int32),
            pltpu.VMEM((128,), jnp.int32),
            pltpu.VMEM((BATCH * N_BUCKETS,), jnp.int32),
            pltpu.VMEM((N_HASHES, 128), jnp.int32),
            pltpu.VMEM((128,), jnp.int32),
            pltpu.VMEM((2, 128, DIM), jnp.float32),
            pltpu.VMEM_SHARED((SEQ, DIM), jnp.float32),
            pltpu.SemaphoreType.DMA,
            pltpu.SemaphoreType.DMA,
        ],
    )(_gather_body)
    return fn(outs_flat, bkt_ht, rank_ht, offs_e.reshape(-1))


def kernel(qk, v, random_rotations):
    rot2 = jnp.transpose(random_rotations, (1, 2, 0))
    qvp, bkt_ht, rank_ht, offs = _run_hash(qk, v, rot2)
    offs_e = offs.reshape(BATCH, N_BUCKETS)
    qvs = _run_scatter(qvp.reshape(BATCH * SEQ, DIM),
                       bkt_ht, rank_ht, offs_e)
    outs = _run_attn(qvs.reshape(BATCH, N_CHUNKS, CHUNK, DIM), _band_mask())
    out = _run_gather(outs.reshape(BATCH * TOTAL, DIM),
                      bkt_ht, rank_ht, offs_e)
    return out.reshape(BATCH, SEQ, DIM)


# speedup vs baseline: 5.0760x; 1.1576x over previous
import functools

import jax
import jax.numpy as jnp
from jax import lax
from jax.experimental import pallas as pl
from jax.experimental.pallas import tpu as pltpu
from jax.experimental.pallas import tpu_sc as plsc

BATCH = 4
SEQ = 4096
DIM = 128
N_HASHES = 8
N_BUCKETS = 64
CHUNK = 128
N_CHUNKS = SEQ * N_HASHES // CHUNK
TOTAL = SEQ * N_HASHES
TB = 512
NEG = -3.4e38


def _pack_bf16(x):
    xb = x.astype(jnp.bfloat16)
    lo = lax.bitcast_convert_type(xb[:, :64], jnp.uint16).astype(jnp.uint32)
    hi = lax.bitcast_convert_type(xb[:, 64:], jnp.uint16).astype(jnp.uint32)
    return lax.bitcast_convert_type(lo | (hi << 16), jnp.float32)


def _unpack_bf16(p):
    u = lax.bitcast_convert_type(p, jnp.uint32)
    lo = lax.bitcast_convert_type((u & 0xFFFF).astype(jnp.uint16),
                                  jnp.bfloat16)
    hi = lax.bitcast_convert_type((u >> 16).astype(jnp.uint16), jnp.bfloat16)
    return jnp.concatenate([lo, hi], axis=-1)


def _hash_body(qk_ref, v_ref, r_ref, qvp_ref, bkt_ref, rank_ref,
               offs_ref, carry):
    tb = pl.program_id(1)

    @pl.when(tb == 0)
    def _():
        carry[...] = jnp.zeros_like(carry)

    x = qk_ref[0]
    nrm = jnp.maximum(jnp.sqrt(jnp.sum(x * x, axis=1, keepdims=True)), 1e-12)
    qkn = x / nrm
    qvp_ref[0] = jnp.concatenate(
        [_pack_bf16(qkn), _pack_bf16(v_ref[0])], axis=1)

    dn = (((1,), (1,)), ((), ()))
    locT = lax.broadcasted_iota(jnp.int32, (32, TB), 0)
    buckets = []
    for h in range(N_HASHES):
        rhT = lax.dot_general(r_ref[h], qkn, dn,
                              preferred_element_type=jnp.float32)
        mp = jnp.max(rhT, axis=0, keepdims=True)
        mnn = jnp.min(rhT, axis=0, keepdims=True)
        cond = mp >= -mnn
        target = jnp.where(cond, mp, mnn)
        addv = jnp.where(cond, 0, 32)
        idx = jnp.min(jnp.where(rhT == target, locT, 64), axis=0,
                      keepdims=True)
        buckets.append(idx + addv)

    sub64 = lax.broadcasted_iota(jnp.int32, (N_BUCKETS, TB), 0)
    ohs = [(b == sub64).astype(jnp.float32) for b in buckets]
    histtok = ohs[0]
    for oh in ohs[1:]:
        histtok = histtok + oh

    ri = lax.broadcasted_iota(jnp.int32, (TB, TB), 0)
    ci = lax.broadcasted_iota(jnp.int32, (TB, TB), 1)
    triu = (ri < ci).astype(jnp.float32)
    cumeT = jnp.dot(histtok, triu, preferred_element_type=jnp.float32)
    priorT = cumeT + carry[:, 0:1]

    ranks = []
    for h in range(N_HASHES):
        csel = jnp.sum(ohs[h] * priorT, axis=0, keepdims=True)
        intra = jnp.zeros((1, TB), jnp.int32)
        for h2 in range(h):
            intra = intra + (buckets[h2] == buckets[h]).astype(jnp.int32)
        ranks.append(csel.astype(jnp.int32) + intra)

    bkt_ref[0] = jnp.concatenate(buckets, axis=0)
    rank_ref[0] = jnp.concatenate(ranks, axis=0)
    new_carry = carry[:, 0:1] + jnp.sum(histtok, axis=1, keepdims=True)
    carry[:, 0:1] = new_carry
    bi = lax.broadcasted_iota(jnp.int32, (N_BUCKETS, N_BUCKETS), 0)
    bj = lax.broadcasted_iota(jnp.int32, (N_BUCKETS, N_BUCKETS), 1)
    triu64 = (bi < bj).astype(jnp.float32)
    offs_ref[0] = lax.dot_general(
        new_carry, triu64, (((0,), (0,)), ((), ())),
        precision=lax.Precision.HIGHEST,
        preferred_element_type=jnp.float32).astype(jnp.int32)


def _run_hash(qk, v, rotations2):
    grid = (BATCH, SEQ // TB)
    return pl.pallas_call(
        _hash_body,
        grid=grid,
        in_specs=[
            pl.BlockSpec((1, TB, DIM), lambda b, t: (b, t, 0)),
            pl.BlockSpec((1, TB, DIM), lambda b, t: (b, t, 0)),
            pl.BlockSpec((N_HASHES, 32, DIM), lambda b, t: (0, 0, 0)),
        ],
        out_specs=[
            pl.BlockSpec((1, TB, DIM), lambda b, t: (b, t, 0)),
            pl.BlockSpec((1, N_HASHES, TB), lambda b, t: (b, 0, t)),
            pl.BlockSpec((1, N_HASHES, TB), lambda b, t: (b, 0, t)),
            pl.BlockSpec((1, 1, N_BUCKETS), lambda b, t: (b, 0, 0)),
        ],
        out_shape=[
            jax.ShapeDtypeStruct((BATCH, SEQ, DIM), jnp.float32),
            jax.ShapeDtypeStruct((BATCH, N_HASHES, SEQ), jnp.int32),
            jax.ShapeDtypeStruct((BATCH, N_HASHES, SEQ), jnp.int32),
            jax.ShapeDtypeStruct((BATCH, 1, N_BUCKETS), jnp.int32),
        ],
        scratch_shapes=[pltpu.VMEM((N_BUCKETS, 128), jnp.float32)],
        compiler_params=pltpu.CompilerParams(
            dimension_semantics=("arbitrary", "arbitrary")),
    )(qk, v, rotations2)


def _scatter_body(qv_hbm, bkt_hbm, rank_hbm, offs_hbm, qvs_hbm,
                  bktv, rankv, offs, idxh8, qvbuf, ssem):
    wid = lax.axis_index("s") * 2 + lax.axis_index("c")
    pltpu.sync_copy(offs_hbm, offs)
    pltpu.sync_copy(qv_hbm.at[pl.ds(wid * 128, 128)], qvbuf)
    handles = []
    for h in range(N_HASHES):
        pltpu.sync_copy(bkt_hbm.at[h, pl.ds(wid * 128, 128)], bktv)
        pltpu.sync_copy(rank_hbm.at[h, pl.ds(wid * 128, 128)], rankv)
        for g in range(8):
            vb = bktv[pl.ds(g * 16, 16)]
            off = plsc.load_gather(offs, [vb])
            idxh8[h, pl.ds(g * 16, 16)] = (
                off + rankv[pl.ds(g * 16, 16)])
        handles.append(
            pltpu.async_copy(qvbuf, qvs_hbm.at[idxh8.at[h]], ssem))
    for hd in handles:
        hd.wait()


def _run_scatter(qv_b, bkt_b, rank_b, offs_b):
    mesh = plsc.VectorSubcoreMesh(core_axis_name="c", subcore_axis_name="s")
    fn = functools.partial(
        pl.kernel,
        out_type=jax.ShapeDtypeStruct((TOTAL, DIM), jnp.float32),
        mesh=mesh,
        compiler_params=pltpu.CompilerParams(needs_layout_passes=False),
        scratch_types=[
            pltpu.VMEM((128,), jnp.int32),
            pltpu.VMEM((128,), jnp.int32),
            pltpu.VMEM((N_BUCKETS,), jnp.int32),
            pltpu.VMEM((N_HASHES, 128), jnp.int32),
            pltpu.VMEM((128, DIM), jnp.float32),
            pltpu.SemaphoreType.DMA,
        ],
    )(_scatter_body)
    return fn(qv_b, bkt_b, rank_b, offs_b)


AC = 4


def _attn_body(qvc_ref, qvp_ref, mask_ref, o_ref):
    dn = (((1,), (1,)), ((), ()))
    scale = DIM ** -0.5
    qvcat = jnp.concatenate([qvp_ref[0, 0]]
                            + [qvc_ref[0, i] for i in range(AC)], axis=0)
    kcat = _unpack_bf16(qvcat[:, :DIM // 2])
    vcat = _unpack_bf16(qvcat[:, DIM // 2:])
    q = _unpack_bf16(
        qvc_ref[0, :, :, :DIM // 2].reshape(AC * CHUNK, DIM // 2)
    ) * jnp.bfloat16(scale)
    d = lax.dot_general(q, kcat, dn, preferred_element_type=jnp.float32)
    e = jnp.exp(d) * mask_ref[...]
    s = jnp.sum(e, axis=1, keepdims=True)
    o = jnp.dot(e.astype(jnp.bfloat16), vcat,
                preferred_element_type=jnp.float32)
    o_ref[0] = (o * (1.0 / (8.0 * s))).reshape(AC, CHUNK, DIM)


def _band_mask():
    r = jnp.arange(AC * CHUNK)[:, None] // CHUNK
    c = jnp.arange((AC + 1) * CHUNK)[None, :] // CHUNK
    return ((c - r >= 0) & (c - r <= 1)).astype(jnp.float32)


def _run_attn(qvs, mask):
    grid = (qvs.shape[0], N_CHUNKS // AC)
    cur = pl.BlockSpec((1, AC, CHUNK, DIM), lambda b, g: (b, g, 0, 0))
    prev = pl.BlockSpec((1, 1, CHUNK, DIM),
                        lambda b, g: (b, (AC * g + N_CHUNKS - 1) % N_CHUNKS,
                                      0, 0))
    mspec = pl.BlockSpec((AC * CHUNK, (AC + 1) * CHUNK), lambda b, g: (0, 0))
    return pl.pallas_call(
        _attn_body,
        grid=grid,
        in_specs=[cur, prev, mspec],
        out_specs=pl.BlockSpec((1, AC, CHUNK, DIM), lambda b, g: (b, g, 0, 0)),
        out_shape=jax.ShapeDtypeStruct((qvs.shape[0], N_CHUNKS, CHUNK, DIM),
                                       jnp.float32),
        compiler_params=pltpu.CompilerParams(
            dimension_semantics=("parallel", "arbitrary")),
    )(qvs, qvs, mask)


def _gather_body(outs_hbm, bkt_hbm, rank_hbm, offs_hbm, out_hbm,
                 bktv, rankv, offs, idxh8, tokq, gbuf2, spacc, gsem, asem):
    wid = lax.axis_index("s") * 2 + lax.axis_index("c")
    i16 = lax.iota(jnp.int32, 16)
    t0 = wid * 128
    for g in range(8):
        tokq[pl.ds(g * 16, 16)] = i16 + (t0 + g * 16)
    pltpu.sync_copy(offs_hbm, offs)

    def build_idx(h):
        pltpu.sync_copy(bkt_hbm.at[h, pl.ds(wid * 128, 128)], bktv)
        pltpu.sync_copy(rank_hbm.at[h, pl.ds(wid * 128, 128)], rankv)
        for g in range(8):
            vb = bktv[pl.ds(g * 16, 16)]
            off = plsc.load_gather(offs, [vb])
            idxh8[h, pl.ds(g * 16, 16)] = off + rankv[pl.ds(g * 16, 16)]

    build_idx(0)
    gh = [None] * N_HASHES
    gh[0] = pltpu.async_copy(outs_hbm.at[idxh8.at[0]], gbuf2.at[0], gsem)
    adds = []
    for h in range(N_HASHES):
        if h + 1 < N_HASHES:
            if h >= 2:
                adds[h - 2].wait()
            build_idx(h + 1)
            gh[h + 1] = pltpu.async_copy(
                outs_hbm.at[idxh8.at[h + 1]], gbuf2.at[(h + 1) % 2], gsem)
        gh[h].wait()
        if h == 0:
            pltpu.async_copy(gbuf2.at[0], spacc.at[pl.ds(t0, 128)],
                             asem).wait()
        else:
            adds.append(pltpu.async_copy(gbuf2.at[h % 2],
                                         spacc.at[tokq], asem, add=True))
    for hd in adds[max(0, N_HASHES - 3):]:
        hd.wait()
    pltpu.sync_copy(spacc.at[pl.ds(t0, 128)],
                    out_hbm.at[pl.ds(t0, 128)])


def _run_gather(outs_b, bkt_b, rank_b, offs_b):
    mesh = plsc.VectorSubcoreMesh(core_axis_name="c", subcore_axis_name="s")
    fn = functools.partial(
        pl.kernel,
        out_type=jax.ShapeDtypeStruct((SEQ, DIM), jnp.float32),
        mesh=mesh,
        compiler_params=pltpu.CompilerParams(needs_layout_passes=False),
        scratch_types=[
            pltpu.VMEM((128,), jnp.int32),
            pltpu.VMEM((128,), jnp.int32),
            pltpu.VMEM((N_BUCKETS,), jnp.int32),
            pltpu.VMEM((N_HASHES, 128), jnp.int32),
            pltpu.VMEM((128,), jnp.int32),
            pltpu.VMEM((2, 128, DIM), jnp.float32),
            pltpu.VMEM_SHARED((SEQ, DIM), jnp.float32),
            pltpu.SemaphoreType.DMA,
            pltpu.SemaphoreType.DMA,
        ],
    )(_gather_body)
    return fn(outs_b, bkt_b, rank_b, offs_b)


def kernel(qk, v, random_rotations):
    rot2 = jnp.transpose(random_rotations, (1, 2, 0))
    qvp, bkt_ht, rank_ht, offs = _run_hash(qk, v, rot2)
    offs_e = offs.reshape(BATCH, N_BUCKETS)
    mask = _band_mask()
    outs = []
    for b in range(BATCH):
        qvs_b = _run_scatter(qvp[b], bkt_ht[b], rank_ht[b], offs_e[b])
        outs_b = _run_attn(qvs_b.reshape(1, N_CHUNKS, CHUNK, DIM), mask)
        outs.append(_run_gather(outs_b.reshape(TOTAL, DIM),
                                bkt_ht[b], rank_ht[b], offs_e[b]))
    return jnp.stack(outs)
